# baseline (speedup 1.0000x reference)
import functools

import jax
import jax.numpy as jnp
from jax import lax
from jax.experimental import pallas as pl
from jax.experimental.pallas import tpu as pltpu
from jax.experimental.pallas import tpu_sc as plsc

B, T, K, D = 8, 576, 8192, 256
R = B * T
NW = 32
RPW = R // NW
L = 16
CH = K // L
GCH = 72
BIG = 2**30


def _snap_body(x_hbm, tab_hbm, out_hbm, buf, idx_v, rows_v, sem0, sem1, semg):
    wid = lax.axis_index("s") * 2 + lax.axis_index("c")
    base = wid * RPW
    iota = lax.iota(jnp.int32, L)
    sems = (sem0, sem1)

    def start_row(r, b):
        @pl.when(r < RPW)
        def _():
            pltpu.async_copy(x_hbm.at[base + r], buf.at[b], sems[b])

    def wait_row(b):
        pltpu.make_async_copy(x_hbm.at[base], buf.at[b], sems[b]).wait()

    start_row(0, 0)
    start_row(1, 1)

    def argmin_row(b):
        def step(i, carry):
            minv, mini = carry
            v = buf[b, pl.ds(i * L, L)]
            c = v < minv
            minv = jnp.where(c, v, minv)
            mini = jnp.where(c, jnp.full((L,), i, jnp.int32), mini)
            return minv, mini

        minv0 = jnp.full((L,), jnp.inf, jnp.float32)
        mini0 = jnp.zeros((L,), jnp.int32)
        minv, mini = lax.fori_loop(0, CH, step, (minv0, mini0))
        m = jnp.min(minv)
        cand = jnp.where(minv == m, mini * L + iota, jnp.full((L,), BIG, jnp.int32))
        return jnp.min(cand)

    def row_pair(r0, _):
        for bslot in range(2):
            r = r0 * 2 + bslot
            wait_row(bslot)
            pos = argmin_row(bslot)
            start_row(r + 2, bslot)
            plsc.store_scatter(
                idx_v,
                [jnp.full((L,), r, jnp.int32)],
                jnp.full((L,), pos, jnp.int32),
                mask=iota == 0,
            )
        return 0

    lax.fori_loop(0, RPW // 2, row_pair, 0)

    copies = [
        pltpu.async_copy(
            tab_hbm.at[idx_v.at[pl.ds(j * GCH, GCH)]],
            rows_v.at[pl.ds(j * GCH, GCH)],
            semg,
        )
        for j in range(RPW // GCH)
    ]
    for cp in copies:
        cp.wait()
    pltpu.sync_copy(rows_v, out_hbm.at[pl.ds(base, RPW)])


@jax.jit
def _snap(x2d, tab):
    mesh = plsc.VectorSubcoreMesh(core_axis_name="c", subcore_axis_name="s")
    return pl.kernel(
        _snap_body,
        out_type=jax.ShapeDtypeStruct((R, D), jnp.float32),
        mesh=mesh,
        compiler_params=pltpu.CompilerParams(needs_layout_passes=False),
        scratch_types=[
            pltpu.VMEM((2, K), jnp.float32),
            pltpu.VMEM((RPW,), jnp.int32),
            pltpu.VMEM((RPW, D), jnp.float32),
            pltpu.SemaphoreType.DMA,
            pltpu.SemaphoreType.DMA,
            pltpu.SemaphoreType.DMA,
        ],
    )(x2d, tab)


def kernel(inputs, reference_table):
    out = _snap(inputs.reshape(R, K), reference_table)
    return out.reshape(B, T, D)


# speedup vs baseline: 3.2975x; 3.2975x over previous
import functools

import jax
import jax.numpy as jnp
from jax import lax
from jax.experimental import pallas as pl
from jax.experimental.pallas import tpu as pltpu
from jax.experimental.pallas import tpu_sc as plsc

B, T, K, D = 8, 576, 8192, 256
R = B * T
NW = 32
RPW = R // NW
L = 16
CH = K // L
GCH = 72
U = 8
BIG = 2**30


def _snap_body(x_hbm, tab_hbm, out_hbm, buf, idx_v, rows_v, sem0, sem1, semg):
    wid = lax.axis_index("s") * 2 + lax.axis_index("c")
    base = wid * RPW
    iota = lax.iota(jnp.int32, L)
    sems = (sem0, sem1)

    def start_row(r, b):
        @pl.when(r < RPW)
        def _():
            pltpu.async_copy(x_hbm.at[base + r], buf.at[b], sems[b])

    def wait_row(b):
        pltpu.make_async_copy(x_hbm.at[base], buf.at[b], sems[b]).wait()

    start_row(0, 0)
    start_row(1, 1)

    def argmin_row(b):
        def step(g, carry):
            minvs, minis = carry
            gsplat = jnp.full((L,), g, jnp.int32)
            off = g * (U * L)
            new_v, new_i = [], []
            for j in range(U):
                v = buf[b, pl.ds(off + j * L, L)]
                c = v < minvs[j]
                new_v.append(jnp.where(c, v, minvs[j]))
                new_i.append(jnp.where(c, gsplat, minis[j]))
            return tuple(new_v), tuple(new_i)

        iv = tuple(jnp.full((L,), jnp.inf, jnp.float32) for _ in range(U))
        ii = tuple(jnp.zeros((L,), jnp.int32) for _ in range(U))
        minvs, minis = lax.fori_loop(0, CH // U, step, (iv, ii))
        mv = minvs[0]
        for j in range(1, U):
            mv = jnp.minimum(mv, minvs[j])
        m = jnp.min(mv)
        cand = jnp.full((L,), BIG, jnp.int32)
        for j in range(U):
            cj = jnp.where(
                minvs[j] == m,
                (minis[j] * U + j) * L + iota,
                jnp.full((L,), BIG, jnp.int32),
            )
            cand = jnp.minimum(cand, cj)
        return jnp.min(cand)

    def row_pair(r0, _):
        for bslot in range(2):
            r = r0 * 2 + bslot
            wait_row(bslot)
            pos = argmin_row(bslot)
            start_row(r + 2, bslot)
            plsc.store_scatter(
                idx_v,
                [jnp.full((L,), r, jnp.int32)],
                jnp.full((L,), pos, jnp.int32),
                mask=iota == 0,
            )
        return 0

    lax.fori_loop(0, RPW // 2, row_pair, 0)

    copies = [
        pltpu.async_copy(
            tab_hbm.at[idx_v.at[pl.ds(j * GCH, GCH)]],
            rows_v.at[pl.ds(j * GCH, GCH)],
            semg,
        )
        for j in range(RPW // GCH)
    ]
    for cp in copies:
        cp.wait()
    pltpu.sync_copy(rows_v, out_hbm.at[pl.ds(base, RPW)])


@jax.jit
def _snap(x2d, tab):
    mesh = plsc.VectorSubcoreMesh(core_axis_name="c", subcore_axis_name="s")
    return pl.kernel(
        _snap_body,
        out_type=jax.ShapeDtypeStruct((R, D), jnp.float32),
        mesh=mesh,
        compiler_params=pltpu.CompilerParams(needs_layout_passes=False),
        scratch_types=[
            pltpu.VMEM((2, K), jnp.float32),
            pltpu.VMEM((RPW,), jnp.int32),
            pltpu.VMEM((RPW, D), jnp.float32),
            pltpu.SemaphoreType.DMA,
            pltpu.SemaphoreType.DMA,
            pltpu.SemaphoreType.DMA,
        ],
    )(x2d, tab)


def kernel(inputs, reference_table):
    out = _snap(inputs.reshape(R, K), reference_table)
    return out.reshape(B, T, D)
